# baseline (speedup 1.0000x reference)
import functools

import jax
import jax.numpy as jnp
from jax import lax
from jax.experimental import pallas as pl
from jax.experimental.pallas import tpu as pltpu
from jax.experimental.pallas import tpu_sc as plsc

N = 10000
D = 128
EMB = 128
OUT = 64
E = 320000

NC = 2
NS = 16
NW = NC * NS
EPW = E // NW
ROWS_PT = N // NS
CHUNK = 80
NCHUNK = EPW // CHUNK

_HI = jax.lax.Precision.HIGHEST


def _k1_body(x_r, w1_r, wd_r, wg_r, attn_r, bd_r, b1_r, w2_r, b2_r,
             feat_o, elr_o, attr_o, acc):
    i = pl.program_id(0)
    h = jnp.maximum(jnp.dot(x_r[...], wd_r[...], precision=_HI) + bd_r[...], 0.0)
    f = jnp.dot(h, wg_r[...], precision=_HI)
    feat_o[...] = f
    elr_o[...] = jnp.dot(f, attn_r[...], precision=_HI)
    xtw = lax.dot_general(x_r[...], w1_r[...], (((0,), (0,)), ((), ())),
                          precision=_HI)

    @pl.when(i == 0)
    def _():
        acc[...] = jnp.zeros_like(acc)

    acc[...] += xtw

    @pl.when(i == pl.num_programs(0) - 1)
    def _():
        a = jnp.maximum(acc[...] + b1_r[...], 0.0)
        attr_o[...] = jnp.dot(a, w2_r[...], precision=_HI) + b2_r[...]


def _k1(x, W1, Wd, Wg, attn2, bd, b1, W2, b2):
    BM = 1000
    nb = N // BM
    return pl.pallas_call(
        _k1_body,
        grid=(nb,),
        in_specs=[
            pl.BlockSpec((BM, D), lambda i: (i, 0)),
            pl.BlockSpec((BM, EMB), lambda i: (i, 0)),
            pl.BlockSpec((D, EMB), lambda i: (0, 0)),
            pl.BlockSpec((EMB, OUT), lambda i: (0, 0)),
            pl.BlockSpec((OUT, 2), lambda i: (0, 0)),
            pl.BlockSpec((1, EMB), lambda i: (0, 0)),
            pl.BlockSpec((1, EMB), lambda i: (0, 0)),
            pl.BlockSpec((EMB, OUT), lambda i: (0, 0)),
            pl.BlockSpec((1, OUT), lambda i: (0, 0)),
        ],
        out_specs=[
            pl.BlockSpec((BM, OUT), lambda i: (i, 0)),
            pl.BlockSpec((BM, 2), lambda i: (i, 0)),
            pl.BlockSpec((D, OUT), lambda i: (0, 0)),
        ],
        out_shape=[
            jax.ShapeDtypeStruct((N, OUT), jnp.float32),
            jax.ShapeDtypeStruct((N, 2), jnp.float32),
            jax.ShapeDtypeStruct((D, OUT), jnp.float32),
        ],
        scratch_shapes=[pltpu.VMEM((D, EMB), jnp.float32)],
    )(x, W1, Wd, Wg, attn2, bd, b1, W2, b2)


def _k2a_body(elr_hbm, src_hbm, dst_hbm, a_hbm, den_hbm,
              elr_v, src_v, dst_v, a_v, den_v, sem):
    c = lax.axis_index("c")
    s = lax.axis_index("s")
    wid = c * NS + s
    base = wid * EPW
    pltpu.sync_copy(elr_hbm, elr_v)
    pltpu.sync_copy(src_hbm.at[pl.ds(base, EPW)], src_v)
    pltpu.sync_copy(dst_hbm.at[pl.ds(base, EPW)], dst_v)

    def zero(i, _):
        den_v[pl.ds(i * 16, 16)] = jnp.zeros((16,), jnp.float32)
        return 0

    lax.fori_loop(0, N // 16, zero, 0)

    def step(i, _):
        sl = pl.ds(i * 16, 16)
        s16 = src_v[sl]
        d16 = dst_v[sl]
        el = plsc.load_gather(elr_v, [s16 * 2])
        er = plsc.load_gather(elr_v, [d16 * 2 + 1])
        e = el + er
        e = jnp.where(e < 0.0, e * 0.2, e)
        a = jnp.exp(e)
        a_v[sl] = a
        plsc.addupdate_scatter(den_v, [d16], a)
        return 0

    lax.fori_loop(0, EPW // 16, step, 0)
    pltpu.sync_copy(a_v, a_hbm.at[pl.ds(base, EPW)])
    pltpu.sync_copy(den_v, den_hbm.at[wid])


def _k2a(elr_flat, src, dst):
    mesh = plsc.VectorSubcoreMesh(core_axis_name="c", subcore_axis_name="s")
    fn = pl.kernel(
        _k2a_body,
        out_type=[
            jax.ShapeDtypeStruct((E,), jnp.float32),
            jax.ShapeDtypeStruct((NW, N), jnp.float32),
        ],
        mesh=mesh,
        scratch_types=[
            pltpu.VMEM((2 * N,), jnp.float32),
            pltpu.VMEM((EPW,), jnp.int32),
            pltpu.VMEM((EPW,), jnp.int32),
            pltpu.VMEM((EPW,), jnp.float32),
            pltpu.VMEM((N,), jnp.float32),
            pltpu.SemaphoreType.DMA,
        ],
        compiler_params=pltpu.CompilerParams(needs_layout_passes=False,
                                             use_tc_tiling_on_sc=False),
    )
    return fn(elr_flat, src, dst)


def _k2m_body(den_r, inv_o):
    inv_o[...] = 1.0 / (jnp.sum(den_r[...], axis=0, keepdims=True) + 1e-9)


def _k2m(den_p):
    return pl.pallas_call(
        _k2m_body,
        grid=(1,),
        in_specs=[pl.BlockSpec((NW, N), lambda i: (0, 0))],
        out_specs=pl.BlockSpec((1, N), lambda i: (0, 0)),
        out_shape=jax.ShapeDtypeStruct((1, N), jnp.float32),
    )(den_p)


def _k2b_body(a_hbm, src_hbm, dst_hbm, inv_hbm, feat_hbm, emb_hbm,
              a_v, src_v, dst_v, w_buf, inv_v, rows_v, zbuf,
              emb_sh, gsem, ssem):
    c = lax.axis_index("c")
    s = lax.axis_index("s")
    wid = c * NS + s
    base = wid * EPW
    pltpu.sync_copy(inv_hbm, inv_v)
    pltpu.sync_copy(a_hbm.at[pl.ds(base, EPW)], a_v)
    pltpu.sync_copy(src_hbm.at[wid], src_v)
    pltpu.sync_copy(dst_hbm.at[wid], dst_v)

    def zzrow(i, _):
        for k in range(OUT // 16):
            zbuf[i, pl.ds(k * 16, 16)] = jnp.zeros((16,), jnp.float32)
        return 0

    lax.fori_loop(0, ROWS_PT // 5, zzrow, 0)
    for m in range(5):
        pltpu.sync_copy(zbuf, emb_sh.at[pl.ds(s * ROWS_PT + m * (ROWS_PT // 5),
                                              ROWS_PT // 5)])
    plsc.subcore_barrier()

    for p in range(3):
        pltpu.async_copy(feat_hbm.at[src_v.at[p]], rows_v.at[p], gsem.at[p])

    def chunk(j, _):
        par = lax.rem(j, 4)

        @pl.when(j + 3 < NCHUNK)
        def _():
            nxt = lax.rem(j + 3, 4)

            @pl.when(j >= 1)
            def _():
                pltpu.make_async_copy(rows_v.at[nxt],
                                      emb_sh.at[dst_v.at[j - 1]],
                                      ssem.at[nxt]).wait()
            pltpu.async_copy(feat_hbm.at[src_v.at[j + 3]], rows_v.at[nxt],
                             gsem.at[nxt])

        for g in range(CHUNK // 16):
            d16 = dst_v[j, pl.ds(g * 16, 16)]
            sl = pl.ds(j * CHUNK + g * 16, 16)
            w_buf[pl.ds(g * 16, 16)] = (
                a_v[sl] * plsc.load_gather(inv_v, [d16]))

        pltpu.make_async_copy(feat_hbm.at[src_v.at[j]], rows_v.at[par],
                              gsem.at[par]).wait()

        rows2d = rows_v.at[par]

        def scale(g, _):
            w16 = w_buf[pl.ds(g * 16, 16)]
            rid = lax.iota(jnp.int32, 16) + g * 16
            for c in range(OUT):
                cc = jnp.full((16,), c, jnp.int32)
                x16 = plsc.load_gather(rows2d, [rid, cc])
                plsc.store_scatter(rows2d, [rid, cc], x16 * w16)
            return 0

        lax.fori_loop(0, CHUNK // 16, scale, 0)
        pltpu.async_copy(rows_v.at[par], emb_sh.at[dst_v.at[j]],
                         ssem.at[par], add=True)
        return 0

    lax.fori_loop(0, NCHUNK, chunk, 0)
    for p in range(4):
        pltpu.make_async_copy(rows_v.at[p], emb_sh.at[dst_v.at[p]],
                              ssem.at[p]).wait()
    plsc.subcore_barrier()
    pltpu.sync_copy(emb_sh.at[pl.ds(s * ROWS_PT, ROWS_PT)],
                    emb_hbm.at[c, pl.ds(s * ROWS_PT, ROWS_PT)])


def _k2b(a_vals, src2, dst2, inv_denom, feat):
    mesh = plsc.VectorSubcoreMesh(core_axis_name="c", subcore_axis_name="s")
    fn = pl.kernel(
        _k2b_body,
        out_type=jax.ShapeDtypeStruct((NC, N, OUT), jnp.float32),
        mesh=mesh,
        scratch_types=[
            pltpu.VMEM((EPW,), jnp.float32),
            pltpu.VMEM((NCHUNK, CHUNK), jnp.int32),
            pltpu.VMEM((NCHUNK, CHUNK), jnp.int32),
            pltpu.VMEM((CHUNK,), jnp.float32),
            pltpu.VMEM((N,), jnp.float32),
            pltpu.VMEM((4, CHUNK, OUT), jnp.float32),
            pltpu.VMEM((ROWS_PT // 5, OUT), jnp.float32),
            pltpu.VMEM_SHARED((N, OUT), jnp.float32),
            pltpu.SemaphoreType.DMA((4,)),
            pltpu.SemaphoreType.DMA((4,)),
        ],
        compiler_params=pltpu.CompilerParams(needs_layout_passes=False,
                                             use_tc_tiling_on_sc=False),
    )
    return fn(a_vals, src2, dst2, inv_denom, feat)


def _k3_body(emb_p_r, bg_r, attr_r, emb_hi_o, emb_lo_o, xhat_o):
    emb = emb_p_r[0] + emb_p_r[1] + bg_r[...]
    hi = emb.astype(jnp.bfloat16)
    emb_hi_o[...] = hi
    emb_lo_o[...] = (emb - hi.astype(jnp.float32)).astype(jnp.bfloat16)
    xhat_o[...] = lax.dot_general(emb, attr_r[...], (((1,), (1,)), ((), ())),
                                  precision=_HI)


def _k3(emb_p, bg, attr):
    BM = 2000
    nb = N // BM
    return pl.pallas_call(
        _k3_body,
        grid=(nb,),
        in_specs=[
            pl.BlockSpec((NC, BM, OUT), lambda i: (0, i, 0)),
            pl.BlockSpec((1, OUT), lambda i: (0, 0)),
            pl.BlockSpec((D, OUT), lambda i: (0, 0)),
        ],
        out_specs=[
            pl.BlockSpec((BM, OUT), lambda i: (i, 0)),
            pl.BlockSpec((BM, OUT), lambda i: (i, 0)),
            pl.BlockSpec((BM, D), lambda i: (i, 0)),
        ],
        out_shape=[
            jax.ShapeDtypeStruct((N, OUT), jnp.bfloat16),
            jax.ShapeDtypeStruct((N, OUT), jnp.bfloat16),
            jax.ShapeDtypeStruct((N, D), jnp.float32),
        ],
    )(emb_p, bg, attr)


def _k4_body(rows_r, cols_r, a_o, *, bn):
    j = pl.program_id(1)
    dims = (((1,), (1,)), ((), ()))
    cj = cols_r[pl.ds(j * bn, bn), :]
    logits = lax.dot_general(rows_r[...], cj, dims,
                             preferred_element_type=jnp.float32)
    a_o[...] = 0.5 * jnp.tanh(0.5 * logits) + 0.5


def _k4(emb_hi, emb_lo):
    BM = 2000
    BN = 1280
    npad = pl.cdiv(N, BN) * BN
    rows_cat = jnp.concatenate([emb_hi, emb_hi, emb_lo], axis=1)
    cols_cat = jnp.pad(jnp.concatenate([emb_hi, emb_lo, emb_hi], axis=1),
                       ((0, npad - N), (0, 0)))
    K3O = 3 * OUT
    return pl.pallas_call(
        functools.partial(_k4_body, bn=BN),
        grid=(N // BM, npad // BN),
        in_specs=[
            pl.BlockSpec((BM, K3O), lambda i, j: (i, 0)),
            pl.BlockSpec((npad, K3O), lambda i, j: (0, 0)),
        ],
        out_specs=pl.BlockSpec((BM, BN), lambda i, j: (i, j)),
        out_shape=jax.ShapeDtypeStruct((N, N), jnp.float32),
    )(rows_cat, cols_cat)


def kernel(x, edge_index, Wd, bd, Wg, attn_l, attn_r, bg, W1, b1, W2, b2):
    src = edge_index[0]
    dst = edge_index[1]
    attn2 = jnp.stack([attn_l, attn_r], axis=1)
    feat, elr, attr = _k1(x, W1, Wd, Wg, attn2,
                          bd.reshape(1, EMB), b1.reshape(1, EMB),
                          W2, b2.reshape(1, OUT))
    elr_flat = elr.reshape(2 * N)
    a_vals, den_p = _k2a(elr_flat, src, dst)
    inv_denom = _k2m(den_p).reshape(N)
    src2 = src.reshape(NW, NCHUNK, CHUNK)
    dst2 = dst.reshape(NW, NCHUNK, CHUNK)
    emb_p = _k2b(a_vals, src2, dst2, inv_denom, feat)
    emb_hi, emb_lo, X_hat = _k3(emb_p, bg.reshape(1, OUT), attr)
    A_hat = _k4(emb_hi, emb_lo)
    return (A_hat, X_hat)


# speedup vs baseline: 3.0703x; 3.0703x over previous
import functools

import jax
import jax.numpy as jnp
from jax import lax
from jax.experimental import pallas as pl
from jax.experimental.pallas import tpu as pltpu
from jax.experimental.pallas import tpu_sc as plsc

N = 10000
D = 128
EMB = 128
OUT = 64
E = 320000

NC = 2
NS = 16
NW = NC * NS
EPW = E // NW
ROWS_PT = N // NS
CHUNK = 80
NCHUNK = EPW // CHUNK

_HI = jax.lax.Precision.HIGHEST


def _k1_body(x_r, w1_r, wd_r, wg_r, attn_r, bd_r, b1_r, w2_r, b2_r,
             feat_o, elr_o, attr_o, acc):
    i = pl.program_id(0)
    h = jnp.maximum(jnp.dot(x_r[...], wd_r[...], precision=_HI) + bd_r[...], 0.0)
    f = jnp.dot(h, wg_r[...], precision=_HI)
    feat_o[...] = f
    elr_o[...] = jnp.dot(f, attn_r[...], precision=_HI)
    xtw = lax.dot_general(x_r[...], w1_r[...], (((0,), (0,)), ((), ())),
                          precision=_HI)

    @pl.when(i == 0)
    def _():
        acc[...] = jnp.zeros_like(acc)

    acc[...] += xtw

    @pl.when(i == pl.num_programs(0) - 1)
    def _():
        a = jnp.maximum(acc[...] + b1_r[...], 0.0)
        attr_o[...] = jnp.dot(a, w2_r[...], precision=_HI) + b2_r[...]


def _k1(x, W1, Wd, Wg, attn2, bd, b1, W2, b2):
    BM = 1000
    nb = N // BM
    return pl.pallas_call(
        _k1_body,
        grid=(nb,),
        in_specs=[
            pl.BlockSpec((BM, D), lambda i: (i, 0)),
            pl.BlockSpec((BM, EMB), lambda i: (i, 0)),
            pl.BlockSpec((D, EMB), lambda i: (0, 0)),
            pl.BlockSpec((EMB, OUT), lambda i: (0, 0)),
            pl.BlockSpec((OUT, 2), lambda i: (0, 0)),
            pl.BlockSpec((1, EMB), lambda i: (0, 0)),
            pl.BlockSpec((1, EMB), lambda i: (0, 0)),
            pl.BlockSpec((EMB, OUT), lambda i: (0, 0)),
            pl.BlockSpec((1, OUT), lambda i: (0, 0)),
        ],
        out_specs=[
            pl.BlockSpec((BM, OUT), lambda i: (i, 0)),
            pl.BlockSpec((BM, 2), lambda i: (i, 0)),
            pl.BlockSpec((D, OUT), lambda i: (0, 0)),
        ],
        out_shape=[
            jax.ShapeDtypeStruct((N, OUT), jnp.float32),
            jax.ShapeDtypeStruct((N, 2), jnp.float32),
            jax.ShapeDtypeStruct((D, OUT), jnp.float32),
        ],
        scratch_shapes=[pltpu.VMEM((D, EMB), jnp.float32)],
    )(x, W1, Wd, Wg, attn2, bd, b1, W2, b2)


def _k2a_body(elr_hbm, src_hbm, dst_hbm, a_hbm, den_hbm,
              elr_v, src_v, dst_v, a_v, den_v, sem):
    c = lax.axis_index("c")
    s = lax.axis_index("s")
    wid = c * NS + s
    base = wid * EPW
    pltpu.sync_copy(elr_hbm, elr_v)
    pltpu.sync_copy(src_hbm.at[pl.ds(base, EPW)], src_v)
    pltpu.sync_copy(dst_hbm.at[pl.ds(base, EPW)], dst_v)

    def zero(i, _):
        den_v[pl.ds(i * 16, 16)] = jnp.zeros((16,), jnp.float32)
        return 0

    lax.fori_loop(0, N // 16, zero, 0)

    def step(i, _):
        sl = pl.ds(i * 16, 16)
        s16 = src_v[sl]
        d16 = dst_v[sl]
        el = plsc.load_gather(elr_v, [s16 * 2])
        er = plsc.load_gather(elr_v, [d16 * 2 + 1])
        e = el + er
        e = jnp.where(e < 0.0, e * 0.2, e)
        a = jnp.exp(e)
        a_v[sl] = a
        plsc.addupdate_scatter(den_v, [d16], a)
        return 0

    lax.fori_loop(0, EPW // 16, step, 0)
    pltpu.sync_copy(a_v, a_hbm.at[pl.ds(base, EPW)])
    pltpu.sync_copy(den_v, den_hbm.at[wid])


def _k2a(elr_flat, src, dst):
    mesh = plsc.VectorSubcoreMesh(core_axis_name="c", subcore_axis_name="s")
    fn = pl.kernel(
        _k2a_body,
        out_type=[
            jax.ShapeDtypeStruct((E,), jnp.float32),
            jax.ShapeDtypeStruct((NW, N), jnp.float32),
        ],
        mesh=mesh,
        scratch_types=[
            pltpu.VMEM((2 * N,), jnp.float32),
            pltpu.VMEM((EPW,), jnp.int32),
            pltpu.VMEM((EPW,), jnp.int32),
            pltpu.VMEM((EPW,), jnp.float32),
            pltpu.VMEM((N,), jnp.float32),
            pltpu.SemaphoreType.DMA,
        ],
        compiler_params=pltpu.CompilerParams(needs_layout_passes=False,
                                             use_tc_tiling_on_sc=False),
    )
    return fn(elr_flat, src, dst)


def _k2m_body(den_r, inv_o):
    inv_o[...] = 1.0 / (jnp.sum(den_r[...], axis=0, keepdims=True) + 1e-9)


def _k2m(den_p):
    return pl.pallas_call(
        _k2m_body,
        grid=(1,),
        in_specs=[pl.BlockSpec((NW, N), lambda i: (0, 0))],
        out_specs=pl.BlockSpec((1, N), lambda i: (0, 0)),
        out_shape=jax.ShapeDtypeStruct((1, N), jnp.float32),
    )(den_p)


def _k2b_body(a_hbm, src_hbm, dst_hbm, inv_hbm, feat_hbm, emb_hbm,
              a_v, src_v, dst_v, w_buf, inv_v, rows0, rows1, rows2, rows3,
              zbuf, emb_sh, gsem, ssem):
    c = lax.axis_index("c")
    s = lax.axis_index("s")
    wid = c * NS + s
    base = wid * EPW
    pltpu.sync_copy(inv_hbm, inv_v)
    pltpu.sync_copy(a_hbm.at[pl.ds(base, EPW)], a_v)
    pltpu.sync_copy(src_hbm.at[wid], src_v)
    pltpu.sync_copy(dst_hbm.at[wid], dst_v)

    def zzrow(i, _):
        for k in range(OUT // 16):
            zbuf[i, pl.ds(k * 16, 16)] = jnp.zeros((16,), jnp.float32)
        return 0

    lax.fori_loop(0, ROWS_PT // 5, zzrow, 0)
    for m in range(5):
        pltpu.sync_copy(zbuf, emb_sh.at[pl.ds(s * ROWS_PT + m * (ROWS_PT // 5),
                                              ROWS_PT // 5)])
    plsc.subcore_barrier()

    rows = (rows0, rows1, rows2, rows3)
    for p in range(3):
        pltpu.async_copy(feat_hbm.at[src_v.at[p]], rows[p], gsem.at[p])

    def body_one(jc, p):
        nxt = (p + 3) % 4

        @pl.when(jc + 3 < NCHUNK)
        def _():
            @pl.when(jc >= 1)
            def _():
                pltpu.make_async_copy(rows[nxt],
                                      emb_sh.at[dst_v.at[jc - 1]],
                                      ssem.at[nxt]).wait()
            pltpu.async_copy(feat_hbm.at[src_v.at[jc + 3]], rows[nxt],
                             gsem.at[nxt])

        for g in range(CHUNK // 16):
            d16 = dst_v[jc, pl.ds(g * 16, 16)]
            sl = pl.ds(jc * CHUNK + g * 16, 16)
            w_buf[pl.ds(g * 16, 16)] = (
                a_v[sl] * plsc.load_gather(inv_v, [d16]))

        pltpu.make_async_copy(feat_hbm.at[src_v.at[jc]], rows[p],
                              gsem.at[p]).wait()
        for g in range(CHUNK // 16):
            w16 = w_buf[pl.ds(g * 16, 16)]
            for r in range(16):
                ws = w16[r]
                row = g * 16 + r
                for k in range(OUT // 16):
                    sl = pl.ds(k * 16, 16)
                    rows[p][row, sl] = rows[p][row, sl] * ws
        pltpu.async_copy(rows[p], emb_sh.at[dst_v.at[jc]],
                         ssem.at[p], add=True)

    def quad(j4, _):
        for p in range(4):
            body_one(j4 * 4 + p, p)
        return 0

    lax.fori_loop(0, NCHUNK // 4, quad, 0)
    body_one(NCHUNK - 1, 0)
    for p in range(4):
        pltpu.make_async_copy(rows[p], emb_sh.at[dst_v.at[p]],
                              ssem.at[p]).wait()
    plsc.subcore_barrier()
    pltpu.sync_copy(emb_sh.at[pl.ds(s * ROWS_PT, ROWS_PT)],
                    emb_hbm.at[c, pl.ds(s * ROWS_PT, ROWS_PT)])


def _k2b(a_vals, src2, dst2, inv_denom, feat):
    mesh = plsc.VectorSubcoreMesh(core_axis_name="c", subcore_axis_name="s")
    fn = pl.kernel(
        _k2b_body,
        out_type=jax.ShapeDtypeStruct((NC, N, OUT), jnp.float32),
        mesh=mesh,
        scratch_types=[
            pltpu.VMEM((EPW,), jnp.float32),
            pltpu.VMEM((NCHUNK, CHUNK), jnp.int32),
            pltpu.VMEM((NCHUNK, CHUNK), jnp.int32),
            pltpu.VMEM((CHUNK,), jnp.float32),
            pltpu.VMEM((N,), jnp.float32),
            pltpu.VMEM((CHUNK, OUT), jnp.float32),
            pltpu.VMEM((CHUNK, OUT), jnp.float32),
            pltpu.VMEM((CHUNK, OUT), jnp.float32),
            pltpu.VMEM((CHUNK, OUT), jnp.float32),
            pltpu.VMEM((ROWS_PT // 5, OUT), jnp.float32),
            pltpu.VMEM_SHARED((N, OUT), jnp.float32),
            pltpu.SemaphoreType.DMA((4,)),
            pltpu.SemaphoreType.DMA((4,)),
        ],
        compiler_params=pltpu.CompilerParams(needs_layout_passes=False,
                                             use_tc_tiling_on_sc=False),
    )
    return fn(a_vals, src2, dst2, inv_denom, feat)


def _k3_body(emb_p_r, bg_r, attr_r, emb_hi_o, emb_lo_o, xhat_o):
    emb = emb_p_r[0] + emb_p_r[1] + bg_r[...]
    hi = emb.astype(jnp.bfloat16)
    emb_hi_o[...] = hi
    emb_lo_o[...] = (emb - hi.astype(jnp.float32)).astype(jnp.bfloat16)
    xhat_o[...] = lax.dot_general(emb, attr_r[...], (((1,), (1,)), ((), ())),
                                  precision=_HI)


def _k3(emb_p, bg, attr):
    BM = 2000
    nb = N // BM
    return pl.pallas_call(
        _k3_body,
        grid=(nb,),
        in_specs=[
            pl.BlockSpec((NC, BM, OUT), lambda i: (0, i, 0)),
            pl.BlockSpec((1, OUT), lambda i: (0, 0)),
            pl.BlockSpec((D, OUT), lambda i: (0, 0)),
        ],
        out_specs=[
            pl.BlockSpec((BM, OUT), lambda i: (i, 0)),
            pl.BlockSpec((BM, OUT), lambda i: (i, 0)),
            pl.BlockSpec((BM, D), lambda i: (i, 0)),
        ],
        out_shape=[
            jax.ShapeDtypeStruct((N, OUT), jnp.bfloat16),
            jax.ShapeDtypeStruct((N, OUT), jnp.bfloat16),
            jax.ShapeDtypeStruct((N, D), jnp.float32),
        ],
    )(emb_p, bg, attr)


def _k4_body(rows_r, cols_r, a_o, *, bn):
    j = pl.program_id(1)
    dims = (((1,), (1,)), ((), ()))
    cj = cols_r[pl.ds(j * bn, bn), :]
    logits = lax.dot_general(rows_r[...], cj, dims,
                             preferred_element_type=jnp.float32)
    a_o[...] = 0.5 * jnp.tanh(0.5 * logits) + 0.5


def _k4(emb_hi, emb_lo):
    BM = 2000
    BN = 1280
    npad = pl.cdiv(N, BN) * BN
    rows_cat = jnp.concatenate([emb_hi, emb_hi, emb_lo], axis=1)
    cols_cat = jnp.pad(jnp.concatenate([emb_hi, emb_lo, emb_hi], axis=1),
                       ((0, npad - N), (0, 0)))
    K3O = 3 * OUT
    return pl.pallas_call(
        functools.partial(_k4_body, bn=BN),
        grid=(N // BM, npad // BN),
        in_specs=[
            pl.BlockSpec((BM, K3O), lambda i, j: (i, 0)),
            pl.BlockSpec((npad, K3O), lambda i, j: (0, 0)),
        ],
        out_specs=pl.BlockSpec((BM, BN), lambda i, j: (i, j)),
        out_shape=jax.ShapeDtypeStruct((N, N), jnp.float32),
    )(rows_cat, cols_cat)


def kernel(x, edge_index, Wd, bd, Wg, attn_l, attn_r, bg, W1, b1, W2, b2):
    src = edge_index[0]
    dst = edge_index[1]
    attn2 = jnp.stack([attn_l, attn_r], axis=1)
    feat, elr, attr = _k1(x, W1, Wd, Wg, attn2,
                          bd.reshape(1, EMB), b1.reshape(1, EMB),
                          W2, b2.reshape(1, OUT))
    elr_flat = elr.reshape(2 * N)
    a_vals, den_p = _k2a(elr_flat, src, dst)
    inv_denom = _k2m(den_p).reshape(N)
    src2 = src.reshape(NW, NCHUNK, CHUNK)
    dst2 = dst.reshape(NW, NCHUNK, CHUNK)
    emb_p = _k2b(a_vals, src2, dst2, inv_denom, feat)
    emb_hi, emb_lo, X_hat = _k3(emb_p, bg.reshape(1, OUT), attr)
    A_hat = _k4(emb_hi, emb_lo)
    return (A_hat, X_hat)
